# baseline (speedup 1.0000x reference)
import functools

import jax
import jax.numpy as jnp
from jax import lax
from jax.experimental import pallas as pl
from jax.experimental.pallas import tpu as pltpu
from jax.experimental.pallas import tpu_sc as plsc

N = 10000
E = 320000
NHID = 64
NFEAT = 128

NC = 2
NS = 16
NW = NC * NS
CHUNK = 128
NCH = 80
E_PAD = NW * NCH * CHUNK
R_PAD = 10240
RPT = R_PAD // NS
RB = 1024


_SC_MESH = plsc.VectorSubcoreMesh(core_axis_name="c", subcore_axis_name="s")


@functools.partial(
    pl.kernel,
    out_type=jax.ShapeDtypeStruct((NC, R_PAD), jnp.float32),
    mesh=_SC_MESH,
    scratch_types=[
        pltpu.VMEM((NCH, CHUNK), jnp.int32),
        pltpu.VMEM((CHUNK,), jnp.float32),
        pltpu.VMEM_SHARED((R_PAD,), jnp.float32),
    ],
)
def _deg_kernel(dst3, zeros1, degp, didx, ones_v, acc):
    c = lax.axis_index("c")
    s = lax.axis_index("s")
    wid = s * NC + c
    pltpu.sync_copy(dst3.at[wid], didx)
    for i in range(CHUNK // 16):
        ones_v[pl.ds(i * 16, 16)] = jnp.full((16,), 1.0, jnp.float32)
    pltpu.sync_copy(zeros1.at[pl.ds(s * RPT, RPT)], acc.at[pl.ds(s * RPT, RPT)])
    plsc.subcore_barrier()

    def body(j, carry):
        pltpu.sync_copy(ones_v, acc.at[didx.at[j]], add=True)
        return carry

    lax.fori_loop(0, NCH, body, 0)
    plsc.subcore_barrier()
    pltpu.sync_copy(acc.at[pl.ds(s * RPT, RPT)], degp.at[c, pl.ds(s * RPT, RPT)])


NBUF = 4


@functools.partial(
    pl.kernel,
    out_type=jax.ShapeDtypeStruct((NC, R_PAD, NHID), jnp.float32),
    mesh=_SC_MESH,
    scratch_types=[
        pltpu.VMEM((NCH, CHUNK), jnp.int32),
        pltpu.VMEM((NCH, CHUNK), jnp.int32),
        pltpu.VMEM((NBUF, CHUNK, NHID), jnp.float32),
        pltpu.VMEM_SHARED((R_PAD, NHID), jnp.float32),
    ]
    + [pltpu.SemaphoreType.DMA] * (2 * NBUF),
    compiler_params=pltpu.CompilerParams(use_tc_tiling_on_sc=False),
)
def _scatter_kernel(table, src3, dst3, zeros2, part, sidx, didx, rows, acc, *sems):
    gs, ss = sems[:NBUF], sems[NBUF:]
    c = lax.axis_index("c")
    s = lax.axis_index("s")
    wid = s * NC + c
    pltpu.sync_copy(src3.at[wid], sidx)
    for b in range(NBUF):
        pltpu.async_copy(table.at[sidx.at[b]], rows.at[b], gs[b])
    pltpu.sync_copy(dst3.at[wid], didx)
    pltpu.sync_copy(zeros2.at[pl.ds(s * RPT, RPT)], acc.at[pl.ds(s * RPT, RPT)])
    plsc.subcore_barrier()

    def gwait(b):
        pltpu.make_async_copy(table.at[sidx.at[0]], rows.at[b], gs[b]).wait()

    def swait(b):
        pltpu.make_async_copy(rows.at[b], acc.at[didx.at[0]], ss[b]).wait()

    def body(i, carry):
        j0 = NBUF * i
        for b in range(NBUF):
            gwait(b)
            pltpu.async_copy(rows.at[b], acc.at[didx.at[j0 + b]], ss[b], add=True)
        for b in range(NBUF):
            jn = j0 + NBUF + b

            @pl.when(jn < NCH)
            def _():
                swait(b)
                pltpu.async_copy(table.at[sidx.at[jn]], rows.at[b], gs[b])

        return carry

    lax.fori_loop(0, NCH // NBUF, body, 0)
    for b in range(NBUF):
        swait(b)
    plsc.subcore_barrier()
    pltpu.sync_copy(acc.at[pl.ds(s * RPT, RPT)], part.at[c, pl.ds(s * RPT, RPT)])


def _dis_of(dt):
    return lax.rsqrt(1.0 + dt[:, 0:1] + dt[:, 1:2])


def _xs_body(dt_ref, x_ref, xs_ref):
    xs_ref[...] = x_ref[...] * _dis_of(dt_ref[...])


def _layer1_body(dt_ref, p_ref, xs_ref, w_ref, b_ref, hs_ref):
    dis = _dis_of(dt_ref[...])
    a = (p_ref[0] + p_ref[1] + xs_ref[...]) * dis
    h = jnp.dot(a, w_ref[...], preferred_element_type=jnp.float32) + b_ref[...]
    hs_ref[...] = jnp.maximum(h, 0.0) * dis


def _layer2_body(dt_ref, q_ref, hs_ref, w_ref, b_ref, o_ref):
    dis = _dis_of(dt_ref[...])
    a = (q_ref[0] + q_ref[1] + hs_ref[...]) * dis
    o_ref[...] = jnp.dot(a, w_ref[...], preferred_element_type=jnp.float32) + b_ref[...]


_GRID = (R_PAD // RB,)
_DT_SPEC = pl.BlockSpec((RB, 2), lambda i: (i, 0))
_ROW_SPEC = pl.BlockSpec((RB, NHID), lambda i: (i, 0))
_P_SPEC = pl.BlockSpec((2, RB, NHID), lambda i: (0, i, 0))


def _tc_xs(dt, x_pad):
    return pl.pallas_call(
        _xs_body,
        grid=_GRID,
        in_specs=[_DT_SPEC, _ROW_SPEC],
        out_specs=_ROW_SPEC,
        out_shape=jax.ShapeDtypeStruct((R_PAD, NHID), jnp.float32),
    )(dt, x_pad)


def _tc_layer1(dt, p, xs, w1, b1):
    return pl.pallas_call(
        _layer1_body,
        grid=_GRID,
        in_specs=[
            _DT_SPEC,
            _P_SPEC,
            _ROW_SPEC,
            pl.BlockSpec((NHID, NHID), lambda i: (0, 0)),
            pl.BlockSpec((1, NHID), lambda i: (0, 0)),
        ],
        out_specs=_ROW_SPEC,
        out_shape=jax.ShapeDtypeStruct((R_PAD, NHID), jnp.float32),
    )(dt, p, xs, w1, b1)


def _tc_layer2(dt, q, hs, w2, b2):
    return pl.pallas_call(
        _layer2_body,
        grid=_GRID,
        in_specs=[
            _DT_SPEC,
            _P_SPEC,
            _ROW_SPEC,
            pl.BlockSpec((NHID, NFEAT), lambda i: (0, 0)),
            pl.BlockSpec((1, NFEAT), lambda i: (0, 0)),
        ],
        out_specs=pl.BlockSpec((RB, NFEAT), lambda i: (i, 0)),
        out_shape=jax.ShapeDtypeStruct((R_PAD, NFEAT), jnp.float32),
    )(dt, q, hs, w2, b2)


def kernel(x, edge_index, W1, b1, W2, b2):
    src = edge_index[0]
    dst = edge_index[1]
    pad = E_PAD - E
    ar = jnp.arange(pad, dtype=jnp.int32)
    src3 = jnp.concatenate([src, ar % N]).reshape(NW, NCH, CHUNK)
    dst3 = jnp.concatenate([dst, N + ar % (R_PAD - N)]).reshape(NW, NCH, CHUNK)
    zeros1 = jnp.zeros((R_PAD,), jnp.float32)
    zeros2 = jnp.zeros((R_PAD, NHID), jnp.float32)
    x_pad = jnp.concatenate([x, jnp.zeros((R_PAD - N, NHID), jnp.float32)])

    degp = _deg_kernel(dst3, zeros1)
    dt = degp.T
    xs = _tc_xs(dt, x_pad)
    p = _scatter_kernel(xs, src3, dst3, zeros2)
    hs = _tc_layer1(dt, p, xs, W1, b1.reshape(1, NHID))
    q = _scatter_kernel(hs, src3, dst3, zeros2)
    out_pad = _tc_layer2(dt, q, hs, W2, b2.reshape(1, NFEAT))
    return out_pad[:N]


# speedup vs baseline: 1.0625x; 1.0625x over previous
import functools

import jax
import jax.numpy as jnp
from jax import lax
from jax.experimental import pallas as pl
from jax.experimental.pallas import tpu as pltpu
from jax.experimental.pallas import tpu_sc as plsc

N = 10000
E = 320000
NHID = 64
NFEAT = 128

NC = 2
NS = 16
NW = NC * NS
CHUNK = 128
NCH = 80
E_PAD = NW * NCH * CHUNK
R_PAD = 10240
RPT = R_PAD // NS
RB = 1000


_SC_MESH = plsc.VectorSubcoreMesh(core_axis_name="c", subcore_axis_name="s")


@functools.partial(
    pl.kernel,
    out_type=jax.ShapeDtypeStruct((NC, R_PAD), jnp.float32),
    mesh=_SC_MESH,
    scratch_types=[
        pltpu.VMEM((NCH, CHUNK), jnp.int32),
        pltpu.VMEM((CHUNK,), jnp.float32),
        pltpu.VMEM_SHARED((R_PAD,), jnp.float32),
    ]
    + [pltpu.SemaphoreType.DMA] * 8,
    compiler_params=pltpu.CompilerParams(use_tc_tiling_on_sc=False),
)
def _deg_kernel(dst3, zeros1, degp, didx, ones_v, acc, *sems):
    c = lax.axis_index("c")
    s = lax.axis_index("s")
    wid = s * NC + c
    pltpu.sync_copy(dst3.at[wid], didx)
    for i in range(CHUNK // 16):
        ones_v[pl.ds(i * 16, 16)] = jnp.full((16,), 1.0, jnp.float32)
    pltpu.sync_copy(zeros1.at[pl.ds(s * RPT, RPT)], acc.at[pl.ds(s * RPT, RPT)])
    plsc.subcore_barrier()

    nsem = len(sems)
    for b in range(nsem):
        pltpu.async_copy(ones_v, acc.at[didx.at[b]], sems[b], add=True)

    def body(i, carry):
        j0 = nsem * i
        for b in range(nsem):
            pltpu.make_async_copy(ones_v, acc.at[didx.at[0]], sems[b]).wait()
            pltpu.async_copy(ones_v, acc.at[didx.at[j0 + nsem + b]], sems[b], add=True)
        return carry

    lax.fori_loop(0, NCH // nsem - 1, body, 0)
    for b in range(nsem):
        pltpu.make_async_copy(ones_v, acc.at[didx.at[0]], sems[b]).wait()
    plsc.subcore_barrier()
    pltpu.sync_copy(acc.at[pl.ds(s * RPT, RPT)], degp.at[c, pl.ds(s * RPT, RPT)])


NBUF = 8


@functools.partial(
    pl.kernel,
    out_type=jax.ShapeDtypeStruct((NC, R_PAD, NHID), jnp.float32),
    mesh=_SC_MESH,
    scratch_types=[
        pltpu.VMEM((NCH, CHUNK), jnp.int32),
        pltpu.VMEM((NCH, CHUNK), jnp.int32),
        pltpu.VMEM((NBUF, CHUNK, NHID), jnp.float32),
        pltpu.VMEM_SHARED((R_PAD, NHID), jnp.float32),
    ]
    + [pltpu.SemaphoreType.DMA] * (2 * NBUF),
    compiler_params=pltpu.CompilerParams(use_tc_tiling_on_sc=False),
)
def _scatter_kernel(table, src3, dst3, zeros2, part, sidx, didx, rows, acc, *sems):
    gs, ss = sems[:NBUF], sems[NBUF:]
    c = lax.axis_index("c")
    s = lax.axis_index("s")
    wid = s * NC + c
    pltpu.sync_copy(src3.at[wid], sidx)
    for b in range(NBUF):
        pltpu.async_copy(table.at[sidx.at[b]], rows.at[b], gs[b])
    pltpu.sync_copy(dst3.at[wid], didx)
    pltpu.sync_copy(zeros2.at[pl.ds(s * RPT, RPT)], acc.at[pl.ds(s * RPT, RPT)])
    plsc.subcore_barrier()

    def gwait(b):
        pltpu.make_async_copy(table.at[sidx.at[0]], rows.at[b], gs[b]).wait()

    def swait(b):
        pltpu.make_async_copy(rows.at[b], acc.at[didx.at[0]], ss[b]).wait()

    def body(i, carry):
        j0 = NBUF * i
        for b in range(NBUF):
            gwait(b)
            pltpu.async_copy(rows.at[b], acc.at[didx.at[j0 + b]], ss[b], add=True)
        for b in range(NBUF):
            jn = j0 + NBUF + b

            @pl.when(jn < NCH)
            def _():
                swait(b)
                pltpu.async_copy(table.at[sidx.at[jn]], rows.at[b], gs[b])

        return carry

    lax.fori_loop(0, NCH // NBUF, body, 0)
    for b in range(NBUF):
        swait(b)
    plsc.subcore_barrier()
    pltpu.sync_copy(acc.at[pl.ds(s * RPT, RPT)], part.at[c, pl.ds(s * RPT, RPT)])


def _dis_of(dt):
    return lax.rsqrt(1.0 + dt[:, 0:1] + dt[:, 1:2])


def _xs_body(dt_ref, x_ref, xs_ref):
    xs_ref[...] = x_ref[...] * _dis_of(dt_ref[...])


def _layer1_body(dt_ref, p_ref, xs_ref, w_ref, b_ref, hs_ref):
    dis = _dis_of(dt_ref[...])
    a = (p_ref[0] + p_ref[1] + xs_ref[...]) * dis
    h = jnp.dot(a, w_ref[...], preferred_element_type=jnp.float32) + b_ref[...]
    hs_ref[...] = jnp.maximum(h, 0.0) * dis


def _layer2_body(dt_ref, q_ref, hs_ref, w_ref, b_ref, o_ref):
    dis = _dis_of(dt_ref[...])
    a = (q_ref[0] + q_ref[1] + hs_ref[...]) * dis
    o_ref[...] = jnp.dot(a, w_ref[...], preferred_element_type=jnp.float32) + b_ref[...]


_GRID = (N // RB,)
_DT_SPEC = pl.BlockSpec((RB, 2), lambda i: (i, 0))
_ROW_SPEC = pl.BlockSpec((RB, NHID), lambda i: (i, 0))
_P_SPEC = pl.BlockSpec((2, RB, NHID), lambda i: (0, i, 0))


def _tc_xs(dt, x):
    return pl.pallas_call(
        _xs_body,
        grid=_GRID,
        in_specs=[_DT_SPEC, _ROW_SPEC],
        out_specs=_ROW_SPEC,
        out_shape=jax.ShapeDtypeStruct((N, NHID), jnp.float32),
    )(dt, x)


def _tc_layer1(dt, p, xs, w1, b1):
    return pl.pallas_call(
        _layer1_body,
        grid=_GRID,
        in_specs=[
            _DT_SPEC,
            _P_SPEC,
            _ROW_SPEC,
            pl.BlockSpec((NHID, NHID), lambda i: (0, 0)),
            pl.BlockSpec((1, NHID), lambda i: (0, 0)),
        ],
        out_specs=_ROW_SPEC,
        out_shape=jax.ShapeDtypeStruct((N, NHID), jnp.float32),
    )(dt, p, xs, w1, b1)


def _tc_layer2(dt, q, hs, w2, b2):
    return pl.pallas_call(
        _layer2_body,
        grid=_GRID,
        in_specs=[
            _DT_SPEC,
            _P_SPEC,
            _ROW_SPEC,
            pl.BlockSpec((NHID, NFEAT), lambda i: (0, 0)),
            pl.BlockSpec((1, NFEAT), lambda i: (0, 0)),
        ],
        out_specs=pl.BlockSpec((RB, NFEAT), lambda i: (i, 0)),
        out_shape=jax.ShapeDtypeStruct((N, NFEAT), jnp.float32),
    )(dt, q, hs, w2, b2)


def kernel(x, edge_index, W1, b1, W2, b2):
    src = edge_index[0]
    dst = edge_index[1]
    pad = E_PAD - E
    ar = jnp.arange(pad, dtype=jnp.int32)
    src3 = jnp.concatenate([src, ar % N]).reshape(NW, NCH, CHUNK)
    dst3 = jnp.concatenate([dst, N + ar % (R_PAD - N)]).reshape(NW, NCH, CHUNK)
    zeros1 = jnp.zeros((R_PAD,), jnp.float32)
    zeros2 = jnp.zeros((R_PAD, NHID), jnp.float32)

    degp = _deg_kernel(dst3, zeros1)
    dt = degp.T
    xs = _tc_xs(dt, x)
    p = _scatter_kernel(xs, src3, dst3, zeros2)
    hs = _tc_layer1(dt, p, xs, W1, b1.reshape(1, NHID))
    q = _scatter_kernel(hs, src3, dst3, zeros2)
    return _tc_layer2(dt, q, hs, W2, b2.reshape(1, NFEAT))
